# baseline (speedup 1.0000x reference)
import functools

import jax
import jax.numpy as jnp
from jax import lax
from jax.experimental import pallas as pl
from jax.experimental.pallas import tpu as pltpu
from jax.experimental.pallas import tpu_sc as plsc

M = 32
K = 256
D = 4
F = 128
B = 16384
NW = 32
RPW = B // NW
RB = 8
NCH = K // 16


def _g16(v, idx):
    return lax.gather(
        v, idx[:, None],
        lax.GatherDimensionNumbers(offset_dims=(), collapsed_slice_dims=(0,),
                                   start_index_map=(0,)),
        (1,), mode=lax.GatherScatterMode.PROMISE_IN_BOUNDS)


def _allsum(v, lane):
    for sh in (1, 2, 4, 8):
        v = v + _g16(v, lane ^ sh)
    return v


def _rsqrt16(s):
    i = lax.bitcast_convert_type(s, jnp.int32)
    i = jnp.int32(0x5F3759DF) - lax.shift_right_logical(i, 1)
    y = lax.bitcast_convert_type(i, jnp.float32)
    for _ in range(3):
        y = y * (1.5 - 0.5 * s * y * y)
    return y


def _sc_body(x_hbm, ct_hbm, xhat_hbm, codes_hbm,
             ct_v, x_v, xn_v, codes_v, xhat_v):
    wid = lax.axis_index("s") * 2 + lax.axis_index("c")
    base = wid * RPW
    pltpu.sync_copy(ct_hbm, ct_v)
    lane = lax.iota(jnp.int32, 16)
    p1 = lane ^ 1
    p2 = lane ^ 2

    NR = 2

    def row_body(rp, _):
        rows = [rp * NR + j for j in range(NR)]
        zero = jnp.zeros((16,), jnp.float32)
        for j, r in enumerate(rows):
            for v in range(F // 16):
                xv = x_v[r, pl.ds(16 * v, 16)]
                s = xv * xv
                s = s + _g16(s, p1)
                s = s + _g16(s, p2)
                s = jnp.maximum(s, 1e-24)
                xn_v[pl.ds(pl.multiple_of((j * F // 16 + v) * 16, 16), 16)] = (
                    xv * _rsqrt16(s))
                xhat_v[r, pl.ds(16 * v, 16)] = zero

        def m_body(m, _):
            vsel = m // 4
            lb = 4 * (m % 4)
            xb = []
            for j in range(NR):
                off = pl.multiple_of((j * (F // 16) + vsel) * 16, 16)
                xnv = xn_v[pl.ds(off, 16)]
                xb.append([_g16(xnv, jnp.full((16,), lb + d, jnp.int32))
                           for d in range(D)])
            ssum = [jnp.zeros((16,), jnp.float32) for _ in range(NR)]
            for ch in range(NCH):
                col = pl.ds(16 * ch, 16)
                ct = [ct_v[4 * m + d, col] for d in range(D)]
                for j, r in enumerate(rows):
                    acc = xb[j][0] * ct[0]
                    acc = acc + xb[j][1] * ct[1]
                    acc = acc + xb[j][2] * ct[2]
                    acc = acc + xb[j][3] * ct[3]
                    e = jnp.exp(acc)
                    codes_v[r, m, col] = e
                    ssum[j] = ssum[j] + e
            rinv = [1.0 / _allsum(ssum[j], lane) for j in range(NR)]
            h = [[jnp.zeros((16,), jnp.float32) for _ in range(D)]
                 for _ in range(NR)]
            for ch in range(NCH):
                col = pl.ds(16 * ch, 16)
                ct = [ct_v[4 * m + d, col] for d in range(D)]
                for j, r in enumerate(rows):
                    cvec = codes_v[r, m, col] * rinv[j]
                    codes_v[r, m, col] = cvec
                    for d in range(D):
                        h[j][d] = h[j][d] + cvec * ct[d]
            hcol = pl.ds(pl.multiple_of(vsel * 16, 16), 16)
            for j, r in enumerate(rows):
                w = jnp.zeros((16,), jnp.float32)
                for d in range(D):
                    w = jnp.where(lane == lb + d, _allsum(h[j][d], lane), w)
                xhat_v[r, hcol] = xhat_v[r, hcol] + w
            return 0

        lax.fori_loop(0, M, m_body, 0)
        return 0

    def chunk_body(g, _):
        row0 = base + g * RB
        pltpu.sync_copy(x_hbm.at[pl.ds(row0, RB), :], x_v)
        lax.fori_loop(0, RB // NR, row_body, 0)
        pltpu.sync_copy(codes_v, codes_hbm.at[pl.ds(row0, RB)])
        pltpu.sync_copy(xhat_v, xhat_hbm.at[pl.ds(row0, RB), :])
        return 0

    lax.fori_loop(0, RPW // RB, chunk_body, 0)


def sc_pq(x, C):
    ct = jnp.transpose(C, (0, 2, 1)).reshape(F, K)
    mesh = plsc.VectorSubcoreMesh(core_axis_name="c", subcore_axis_name="s")
    run = functools.partial(
        pl.kernel, mesh=mesh,
        out_type=[jax.ShapeDtypeStruct((B, F), jnp.float32),
                  jax.ShapeDtypeStruct((B, M, K), jnp.float32)],
        scratch_types=[
            pltpu.VMEM((F, K), jnp.float32),
            pltpu.VMEM((RB, F), jnp.float32),
            pltpu.VMEM((2 * F,), jnp.float32),
            pltpu.VMEM((RB, M, K), jnp.float32),
            pltpu.VMEM((RB, F), jnp.float32),
        ],
    )(_sc_body)
    xhat, codes = run(x, ct)
    return xhat, codes


def kernel(x, C):
    return sc_pq(x, C)


# speedup vs baseline: 1.2877x; 1.2877x over previous
import functools

import jax
import jax.numpy as jnp
from jax import lax
from jax.experimental import pallas as pl
from jax.experimental.pallas import tpu as pltpu
from jax.experimental.pallas import tpu_sc as plsc

M = 32
K = 256
D = 4
F = 128
B = 16384
NW = 32
RPW = B // NW
RB = 8
NCH = K // 16


def _g16(v, idx):
    return lax.gather(
        v, idx[:, None],
        lax.GatherDimensionNumbers(offset_dims=(), collapsed_slice_dims=(0,),
                                   start_index_map=(0,)),
        (1,), mode=lax.GatherScatterMode.PROMISE_IN_BOUNDS)


def _allsum(v, lane):
    for sh in (1, 2, 4, 8):
        v = v + _g16(v, lane ^ sh)
    return v


def _rsqrt16(s):
    i = lax.bitcast_convert_type(s, jnp.int32)
    i = jnp.int32(0x5F3759DF) - lax.shift_right_logical(i, 1)
    y = lax.bitcast_convert_type(i, jnp.float32)
    for _ in range(3):
        y = y * (1.5 - 0.5 * s * y * y)
    return y


def _sc_body(x_hbm, ct_hbm, xhat_hbm, codes_hbm,
             ct_v, x_v, xn_v, codes_v, xhat_v):
    wid = lax.axis_index("s") * 2 + lax.axis_index("c")
    base = wid * RPW
    pltpu.sync_copy(ct_hbm, ct_v)
    lane = lax.iota(jnp.int32, 16)
    p1 = lane ^ 1
    p2 = lane ^ 2

    NR = 4

    def row_body(rp, _):
        rows = [rp * NR + j for j in range(NR)]
        zero = jnp.zeros((16,), jnp.float32)
        for j, r in enumerate(rows):
            for v in range(F // 16):
                xv = x_v[r, pl.ds(16 * v, 16)]
                s = xv * xv
                s = s + _g16(s, p1)
                s = s + _g16(s, p2)
                s = jnp.maximum(s, 1e-24)
                xn_v[pl.ds(pl.multiple_of((j * F // 16 + v) * 16, 16), 16)] = (
                    xv * _rsqrt16(s))
                xhat_v[r, pl.ds(16 * v, 16)] = zero

        def m_body(m, _):
            vsel = m // 4
            lb = 4 * (m % 4)
            xb = []
            for j in range(NR):
                off = pl.multiple_of((j * (F // 16) + vsel) * 16, 16)
                xnv = xn_v[pl.ds(off, 16)]
                xb.append([_g16(xnv, jnp.full((16,), lb + d, jnp.int32))
                           for d in range(D)])
            ssum = [jnp.zeros((16,), jnp.float32) for _ in range(NR)]
            for ch in range(NCH):
                col = pl.ds(16 * ch, 16)
                ct = [ct_v[4 * m + d, col] for d in range(D)]
                for j, r in enumerate(rows):
                    acc = xb[j][0] * ct[0]
                    acc = acc + xb[j][1] * ct[1]
                    acc = acc + xb[j][2] * ct[2]
                    acc = acc + xb[j][3] * ct[3]
                    e = jnp.exp(acc)
                    codes_v[r, m, col] = e
                    ssum[j] = ssum[j] + e
            rinv = [1.0 / _allsum(ssum[j], lane) for j in range(NR)]
            h = [[jnp.zeros((16,), jnp.float32) for _ in range(D)]
                 for _ in range(NR)]
            for ch in range(NCH):
                col = pl.ds(16 * ch, 16)
                ct = [ct_v[4 * m + d, col] for d in range(D)]
                for j, r in enumerate(rows):
                    cvec = codes_v[r, m, col] * rinv[j]
                    codes_v[r, m, col] = cvec
                    for d in range(D):
                        h[j][d] = h[j][d] + cvec * ct[d]
            hcol = pl.ds(pl.multiple_of(vsel * 16, 16), 16)
            for j, r in enumerate(rows):
                w = jnp.zeros((16,), jnp.float32)
                for d in range(D):
                    w = jnp.where(lane == lb + d, _allsum(h[j][d], lane), w)
                xhat_v[r, hcol] = xhat_v[r, hcol] + w
            return 0

        lax.fori_loop(0, M, m_body, 0)
        return 0

    def chunk_body(g, _):
        row0 = base + g * RB
        pltpu.sync_copy(x_hbm.at[pl.ds(row0, RB), :], x_v)
        lax.fori_loop(0, RB // NR, row_body, 0)
        pltpu.sync_copy(codes_v, codes_hbm.at[pl.ds(row0, RB)])
        pltpu.sync_copy(xhat_v, xhat_hbm.at[pl.ds(row0, RB), :])
        return 0

    lax.fori_loop(0, RPW // RB, chunk_body, 0)


def sc_pq(x, C):
    ct = jnp.transpose(C, (0, 2, 1)).reshape(F, K)
    mesh = plsc.VectorSubcoreMesh(core_axis_name="c", subcore_axis_name="s")
    run = functools.partial(
        pl.kernel, mesh=mesh,
        out_type=[jax.ShapeDtypeStruct((B, F), jnp.float32),
                  jax.ShapeDtypeStruct((B, M, K), jnp.float32)],
        scratch_types=[
            pltpu.VMEM((F, K), jnp.float32),
            pltpu.VMEM((RB, F), jnp.float32),
            pltpu.VMEM((4 * F,), jnp.float32),
            pltpu.VMEM((RB, M, K), jnp.float32),
            pltpu.VMEM((RB, F), jnp.float32),
        ],
    )(_sc_body)
    xhat, codes = run(x, ct)
    return xhat, codes


def kernel(x, C):
    return sc_pq(x, C)


# speedup vs baseline: 2.0398x; 1.5840x over previous
import functools

import jax
import jax.numpy as jnp
from jax import lax
from jax.experimental import pallas as pl
from jax.experimental.pallas import tpu as pltpu
from jax.experimental.pallas import tpu_sc as plsc

M = 32
K = 256
D = 4
F = 128
B = 16384
NW = 32
RPW = B // NW
RB = 8
NCH = K // 16


def _g16(v, idx):
    return lax.gather(
        v, idx[:, None],
        lax.GatherDimensionNumbers(offset_dims=(), collapsed_slice_dims=(0,),
                                   start_index_map=(0,)),
        (1,), mode=lax.GatherScatterMode.PROMISE_IN_BOUNDS)


def _allsum(v, lane):
    for sh in (1, 2, 4, 8):
        v = v + _g16(v, lane ^ sh)
    return v


def _rsqrt16(s):
    i = lax.bitcast_convert_type(s, jnp.int32)
    i = jnp.int32(0x5F3759DF) - lax.shift_right_logical(i, 1)
    y = lax.bitcast_convert_type(i, jnp.float32)
    for _ in range(3):
        y = y * (1.5 - 0.5 * s * y * y)
    return y


def _sc_body(x_hbm, ct_hbm, xhat_hbm, codes_hbm,
             ct_v, x_v, xn_v, codes_v, xhat_v):
    wid = lax.axis_index("s") * 2 + lax.axis_index("c")
    base = wid * RPW
    pltpu.sync_copy(ct_hbm, ct_v)
    lane = lax.iota(jnp.int32, 16)
    p1 = lane ^ 1
    p2 = lane ^ 2

    NR = 8

    def row_body(rp, _):
        rows = [rp * NR + j for j in range(NR)]
        zero = jnp.zeros((16,), jnp.float32)
        for j, r in enumerate(rows):
            for v in range(F // 16):
                xv = x_v[r, pl.ds(16 * v, 16)]
                s = xv * xv
                s = s + _g16(s, p1)
                s = s + _g16(s, p2)
                s = jnp.maximum(s, 1e-24)
                xn_v[pl.ds(pl.multiple_of((j * F // 16 + v) * 16, 16), 16)] = (
                    xv * _rsqrt16(s))
                xhat_v[r, pl.ds(16 * v, 16)] = zero

        def m_body(m, _):
            vsel = m // 4
            lb = 4 * (m % 4)
            xb = []
            for j in range(NR):
                off = pl.multiple_of((j * (F // 16) + vsel) * 16, 16)
                xnv = xn_v[pl.ds(off, 16)]
                xb.append([_g16(xnv, jnp.full((16,), lb + d, jnp.int32))
                           for d in range(D)])
            ssum = [jnp.zeros((16,), jnp.float32) for _ in range(NR)]
            for ch in range(NCH):
                col = pl.ds(16 * ch, 16)
                ct = [ct_v[4 * m + d, col] for d in range(D)]
                for j, r in enumerate(rows):
                    acc = xb[j][0] * ct[0]
                    acc = acc + xb[j][1] * ct[1]
                    acc = acc + xb[j][2] * ct[2]
                    acc = acc + xb[j][3] * ct[3]
                    e = jnp.exp(acc)
                    codes_v[r, m, col] = e
                    ssum[j] = ssum[j] + e
            rinv = [1.0 / _allsum(ssum[j], lane) for j in range(NR)]
            h = [[jnp.zeros((16,), jnp.float32) for _ in range(D)]
                 for _ in range(NR)]
            for ch in range(NCH):
                col = pl.ds(16 * ch, 16)
                ct = [ct_v[4 * m + d, col] for d in range(D)]
                for j, r in enumerate(rows):
                    cvec = codes_v[r, m, col] * rinv[j]
                    codes_v[r, m, col] = cvec
                    for d in range(D):
                        h[j][d] = h[j][d] + cvec * ct[d]
            hcol = pl.ds(pl.multiple_of(vsel * 16, 16), 16)
            for j, r in enumerate(rows):
                w = jnp.zeros((16,), jnp.float32)
                for d in range(D):
                    w = jnp.where(lane == lb + d, _allsum(h[j][d], lane), w)
                xhat_v[r, hcol] = xhat_v[r, hcol] + w
            return 0

        lax.fori_loop(0, M, m_body, 0)
        return 0

    def chunk_body(g, _):
        row0 = base + g * RB
        pltpu.sync_copy(x_hbm.at[pl.ds(row0, RB), :], x_v)
        lax.fori_loop(0, RB // NR, row_body, 0)
        pltpu.sync_copy(codes_v, codes_hbm.at[pl.ds(row0, RB)])
        pltpu.sync_copy(xhat_v, xhat_hbm.at[pl.ds(row0, RB), :])
        return 0

    lax.fori_loop(0, RPW // RB, chunk_body, 0)


def sc_pq(x, C):
    ct = jnp.transpose(C, (0, 2, 1)).reshape(F, K)
    mesh = plsc.VectorSubcoreMesh(core_axis_name="c", subcore_axis_name="s")
    run = functools.partial(
        pl.kernel, mesh=mesh,
        out_type=[jax.ShapeDtypeStruct((B, F), jnp.float32),
                  jax.ShapeDtypeStruct((B, M, K), jnp.float32)],
        scratch_types=[
            pltpu.VMEM((F, K), jnp.float32),
            pltpu.VMEM((RB, F), jnp.float32),
            pltpu.VMEM((8 * F,), jnp.float32),
            pltpu.VMEM((RB, M, K), jnp.float32),
            pltpu.VMEM((RB, F), jnp.float32),
        ],
    )(_sc_body)
    xhat, codes = run(x, ct)
    return xhat, codes


def kernel(x, C):
    return sc_pq(x, C)


# speedup vs baseline: 14.9205x; 7.3148x over previous
import jax
import jax.numpy as jnp
from jax import lax
from jax.experimental import pallas as pl

M = 32
K = 256
D = 4
F = 128
B = 16384
BB = 256
R = BB * M


def _pq_body(x_ref, cf_ref, cft_ref, mask_ref, xhat_ref, codes_ref):
    x = x_ref[...]
    r = lax.broadcasted_iota(jnp.int32, (F, F), 0) // D
    c = lax.broadcasted_iota(jnp.int32, (F, F), 1) // D
    g = (r == c).astype(jnp.float32)
    ssq = jnp.dot(x * x, g, preferred_element_type=jnp.float32,
                  precision=lax.Precision.HIGHEST)
    inv = lax.rsqrt(jnp.maximum(ssq, 1e-24))
    xn = x * inv
    xrep = jnp.broadcast_to(xn[:, None, :], (BB, M, F)).reshape(R, F)
    xm = (xrep * mask_ref[...]).astype(jnp.bfloat16)
    ips = jnp.dot(xm, cf_ref[...], preferred_element_type=jnp.float32)
    e = jnp.exp(ips)
    s = jnp.dot(e.astype(jnp.bfloat16), jnp.ones((K, 1), jnp.bfloat16),
                preferred_element_type=jnp.float32)
    sb = jnp.broadcast_to(1.0 / s, (R, K))
    codes = e * sb
    codes_ref[...] = codes.reshape(BB, M, K)
    ph = jnp.dot(codes.astype(jnp.bfloat16), cft_ref[...],
                 preferred_element_type=jnp.float32)
    phm = (ph * mask_ref[...]).reshape(BB, M, F)
    xhat_ref[...] = jnp.sum(phm, axis=1)


def kernel(x, C):
    cf = jnp.transpose(C, (0, 2, 1)).reshape(F, K).astype(jnp.bfloat16)
    cft2 = jnp.transpose(C, (1, 0, 2)).reshape(K, F).astype(jnp.bfloat16)
    lane = jnp.arange(F, dtype=jnp.int32) // D
    row = jnp.arange(M, dtype=jnp.int32)
    mask = (lane[None, :] == row[:, None]).astype(jnp.float32)
    mask = jnp.tile(mask, (BB, 1))
    grid = (B // BB,)
    xhat, codes = pl.pallas_call(
        _pq_body,
        grid=grid,
        in_specs=[
            pl.BlockSpec((BB, F), lambda i: (i, 0)),
            pl.BlockSpec((F, K), lambda i: (0, 0)),
            pl.BlockSpec((K, F), lambda i: (0, 0)),
            pl.BlockSpec((R, F), lambda i: (0, 0)),
        ],
        out_specs=[
            pl.BlockSpec((BB, F), lambda i: (i, 0)),
            pl.BlockSpec((BB, M, K), lambda i: (i, 0, 0)),
        ],
        out_shape=[
            jax.ShapeDtypeStruct((B, F), jnp.float32),
            jax.ShapeDtypeStruct((B, M, K), jnp.float32),
        ],
    )(x, cf, cft2, mask)
    return xhat, codes
